# baseline (speedup 1.0000x reference)
import jax
import jax.numpy as jnp
from jax.experimental import pallas as pl
from jax.experimental.pallas import tpu as pltpu

_N = 50000
_D = 1024
_L = 512
_TILE = 1000
_CHUNK = 500
_GRID = _N // _TILE


def _chain(hh, w1t_ref, b1_ref, wabt_ref, bab_ref, wct_ref, bc_ref):
    xp = jnp.dot(hh.astype(jnp.bfloat16), w1t_ref[...],
                 preferred_element_type=jnp.float32)
    x = jnp.maximum(xp + b1_ref[...], 0).astype(jnp.bfloat16)
    ab = jnp.dot(x, wabt_ref[...],
                 preferred_element_type=jnp.float32)
    ab = ab + bab_ref[...]
    a = jnp.tanh(ab[:, :256])
    g = jax.nn.sigmoid(ab[:, 256:])
    att = jnp.dot((a * g).astype(jnp.bfloat16), wct_ref[...],
                  preferred_element_type=jnp.float32)
    att = att + bc_ref[...]
    e = jnp.exp(att)
    return att, e, x


def _clam_body(h_ref, w1t_ref, b1_ref, wabt_ref, bab_ref, wct_ref, bc_ref,
               wcls_ref, bcls_ref,
               a_out_ref, logits_ref, yprob_ref, yhat_ref,
               m_acc, s_acc):
    i = pl.program_id(0)

    @pl.when(i == 0)
    def _init():
        m_acc[...] = jnp.zeros_like(m_acc)
        s_acc[...] = jnp.zeros_like(s_acc)

    chains = [_chain(h_ref[k * _CHUNK:(k + 1) * _CHUNK], w1t_ref,
                     b1_ref, wabt_ref, bab_ref, wct_ref, bc_ref)
              for k in range(_TILE // _CHUNK)]
    for k, (att_k, _, _) in enumerate(chains):
        a_out_ref[k * _CHUNK:(k + 1) * _CHUNK] = att_k

    s_acc[...] += sum(jnp.sum(e_k, axis=0, keepdims=True)
                      for _, e_k, _ in chains)
    m_acc[...] += sum(jax.lax.dot_general(
        e_k.astype(jnp.bfloat16), x_k, (((0,), (0,)), ((), ())),
        preferred_element_type=jnp.float32) for _, e_k, x_k in chains)

    @pl.when(i == _GRID - 1)
    def _final():
        l0 = jnp.sum(m_acc[0:1, :] * wcls_ref[0:1, :], axis=1, keepdims=True)
        l1 = jnp.sum(m_acc[1:2, :] * wcls_ref[1:2, :], axis=1, keepdims=True)
        raw = jnp.concatenate([l0, l1], axis=1)
        logits = raw / s_acc[...] + bcls_ref[...]
        logits_ref[...] = logits
        mx = jnp.max(logits, axis=1, keepdims=True)
        ee = jnp.exp(logits - mx)
        yprob_ref[...] = ee / jnp.sum(ee, axis=1, keepdims=True)
        col = jax.lax.broadcasted_iota(jnp.int32, (1, 2), 1)
        yhat_ref[...] = jnp.min(jnp.where(logits == mx, col, 2),
                                axis=1, keepdims=True)


def kernel(h, W1, b1, Wa, ba, Wb, bb, Wc, bc, Wcls0, bcls0, Wcls1, bcls1):
    w1t = W1.T.astype(jnp.bfloat16)
    wabt = jnp.concatenate([Wa, Wb], axis=0).T.astype(jnp.bfloat16)
    bab = jnp.concatenate([ba, bb])[None, :]
    wct = Wc.T.astype(jnp.bfloat16)
    bcv = bc[None, :]
    wcls = jnp.concatenate([Wcls0, Wcls1], axis=0)
    bcls = jnp.stack([bcls0[0], bcls1[0]])[None, :]

    a_nt, logits, yprob, yhat = pl.pallas_call(
        _clam_body,
        grid=(_GRID,),
        in_specs=[
            pl.BlockSpec((_TILE, _D), lambda i: (i, 0)),
            pl.BlockSpec((_D, _L), lambda i: (0, 0)),
            pl.BlockSpec((1, _L), lambda i: (0, 0)),
            pl.BlockSpec((_L, _L), lambda i: (0, 0)),
            pl.BlockSpec((1, _L), lambda i: (0, 0)),
            pl.BlockSpec((256, 2), lambda i: (0, 0)),
            pl.BlockSpec((1, 2), lambda i: (0, 0)),
            pl.BlockSpec((2, _L), lambda i: (0, 0)),
            pl.BlockSpec((1, 2), lambda i: (0, 0)),
        ],
        out_specs=[
            pl.BlockSpec((_TILE, 2), lambda i: (i, 0)),
            pl.BlockSpec((1, 2), lambda i: (0, 0)),
            pl.BlockSpec((1, 2), lambda i: (0, 0)),
            pl.BlockSpec((1, 1), lambda i: (0, 0)),
        ],
        out_shape=[
            jax.ShapeDtypeStruct((_N, 2), jnp.float32),
            jax.ShapeDtypeStruct((1, 2), jnp.float32),
            jax.ShapeDtypeStruct((1, 2), jnp.float32),
            jax.ShapeDtypeStruct((1, 1), jnp.int32),
        ],
        scratch_shapes=[
            pltpu.VMEM((2, _L), jnp.float32),
            pltpu.VMEM((1, 2), jnp.float32),
        ],
        compiler_params=pltpu.CompilerParams(
            dimension_semantics=("arbitrary",),
        ),
    )(h, w1t, b1[None, :], wabt, bab, wct, bcv, wcls, bcls)

    return (logits, yprob, yhat, a_nt.T)


# speedup vs baseline: 1.2086x; 1.2086x over previous
import jax
import jax.numpy as jnp
from jax.experimental import pallas as pl
from jax.experimental.pallas import tpu as pltpu

_N = 50000
_D = 1024
_L = 512
_TILE = 2000
_GRID = _N // _TILE
_SPLITS = (0, 1024, 2000)


def _chain(hh, w1t_ref, b1_ref, wabt_ref, bab_ref, wct_ref, bc_ref):
    xp = jnp.dot(hh.astype(jnp.bfloat16), w1t_ref[...],
                 preferred_element_type=jnp.float32)
    x = jnp.maximum(xp + b1_ref[...], 0).astype(jnp.bfloat16)
    ab = jnp.dot(x, wabt_ref[...],
                 preferred_element_type=jnp.float32)
    ab = ab + bab_ref[...]
    a = jnp.tanh(ab[:, :256])
    g = jax.nn.sigmoid(ab[:, 256:])
    att_t = jax.lax.dot_general(wct_ref[...], (a * g).astype(jnp.bfloat16),
                                (((1,), (1,)), ((), ())),
                                preferred_element_type=jnp.float32)
    att_t = att_t + bc_ref[...]
    e_t = jnp.exp(att_t)
    return att_t, e_t, x


def _clam_body(h_ref, w1t_ref, b1_ref, wabt_ref, bab_ref, wct_ref, bc_ref,
               wcls_ref, bcls_ref,
               a_out_ref, logits_ref, yprob_ref, yhat_ref,
               m_acc, s_acc):
    i = pl.program_id(0)

    @pl.when(i == 0)
    def _init():
        m_acc[...] = jnp.zeros_like(m_acc)
        s_acc[...] = jnp.zeros_like(s_acc)

    chains = [_chain(h_ref[_SPLITS[k]:_SPLITS[k + 1]], w1t_ref,
                     b1_ref, wabt_ref, bab_ref, wct_ref, bc_ref)
              for k in range(len(_SPLITS) - 1)]
    for k, (att_k, _, _) in enumerate(chains):
        a_out_ref[0, :, _SPLITS[k]:_SPLITS[k + 1]] = att_k

    s_acc[...] += sum(jnp.sum(e_k, axis=1, keepdims=True)
                      for _, e_k, _ in chains)
    m_acc[...] += sum(jnp.dot(
        e_k.astype(jnp.bfloat16), x_k,
        preferred_element_type=jnp.float32) for _, e_k, x_k in chains)

    @pl.when(i == _GRID - 1)
    def _final():
        raw = jnp.sum(m_acc[...] * wcls_ref[...], axis=1, keepdims=True)
        lcol = raw / s_acc[...] + bcls_ref[...]
        logits = jnp.concatenate([lcol[0:1, :], lcol[1:2, :]], axis=1)
        logits_ref[...] = logits
        mx = jnp.max(logits, axis=1, keepdims=True)
        ee = jnp.exp(logits - mx)
        yprob_ref[...] = ee / jnp.sum(ee, axis=1, keepdims=True)
        col = jax.lax.broadcasted_iota(jnp.int32, (1, 2), 1)
        yhat_ref[...] = jnp.min(jnp.where(logits == mx, col, 2),
                                axis=1, keepdims=True)


def kernel(h, W1, b1, Wa, ba, Wb, bb, Wc, bc, Wcls0, bcls0, Wcls1, bcls1):
    w1t = W1.T.astype(jnp.bfloat16)
    wabt = jnp.concatenate([Wa, Wb], axis=0).T.astype(jnp.bfloat16)
    bab = jnp.concatenate([ba, bb])[None, :]
    wct = Wc.astype(jnp.bfloat16)
    bcv = bc[:, None]
    wcls = jnp.concatenate([Wcls0, Wcls1], axis=0)
    bcls = jnp.stack([bcls0[0], bcls1[0]])[:, None]

    a_st, logits, yprob, yhat = pl.pallas_call(
        _clam_body,
        grid=(_GRID,),
        in_specs=[
            pl.BlockSpec((_TILE, _D), lambda i: (i, 0)),
            pl.BlockSpec((_D, _L), lambda i: (0, 0)),
            pl.BlockSpec((1, _L), lambda i: (0, 0)),
            pl.BlockSpec((_L, _L), lambda i: (0, 0)),
            pl.BlockSpec((1, _L), lambda i: (0, 0)),
            pl.BlockSpec((2, 256), lambda i: (0, 0)),
            pl.BlockSpec((2, 1), lambda i: (0, 0)),
            pl.BlockSpec((2, _L), lambda i: (0, 0)),
            pl.BlockSpec((2, 1), lambda i: (0, 0)),
        ],
        out_specs=[
            pl.BlockSpec((1, 2, _TILE), lambda i: (i, 0, 0)),
            pl.BlockSpec((1, 2), lambda i: (0, 0)),
            pl.BlockSpec((1, 2), lambda i: (0, 0)),
            pl.BlockSpec((1, 1), lambda i: (0, 0)),
        ],
        out_shape=[
            jax.ShapeDtypeStruct((_GRID, 2, _TILE), jnp.float32),
            jax.ShapeDtypeStruct((1, 2), jnp.float32),
            jax.ShapeDtypeStruct((1, 2), jnp.float32),
            jax.ShapeDtypeStruct((1, 1), jnp.int32),
        ],
        scratch_shapes=[
            pltpu.VMEM((2, _L), jnp.float32),
            pltpu.VMEM((2, 1), jnp.float32),
        ],
        compiler_params=pltpu.CompilerParams(
            dimension_semantics=("arbitrary",),
        ),
    )(h, w1t, b1[None, :], wabt, bab, wct, bcv, wcls, bcls)

    a_raw = jnp.transpose(a_st, (1, 0, 2)).reshape(2, _N)
    return (logits, yprob, yhat, a_raw)
